# baseline (speedup 1.0000x reference)
import functools

import jax
import jax.numpy as jnp
from jax import lax
from jax.experimental import pallas as pl
from jax.experimental.pallas import tpu as pltpu
from jax.experimental.pallas import tpu_sc as plsc

N, D, H, O, E, K = 2048, 768, 768, 768, 8, 2
BM = 256
T = (N * K) // BM
S = T + E - 1
NW = 32
SLOTS = N * K


def _gating_body(x_ref, wg_ref, bg_ref, w_ref, i_ref):
    logits = jnp.dot(x_ref[...], wg_ref[...],
                     preferred_element_type=jnp.float32) + bg_ref[...]
    cols = lax.broadcasted_iota(jnp.int32, logits.shape, 1)
    m1 = jnp.max(logits, axis=-1, keepdims=True)
    i1 = jnp.min(jnp.where(logits == m1, cols, E), axis=-1, keepdims=True)
    l2 = jnp.where(cols == i1, -jnp.inf, logits)
    m2 = jnp.max(l2, axis=-1, keepdims=True)
    i2 = jnp.min(jnp.where(l2 == m2, cols, E), axis=-1, keepdims=True)
    e2 = jnp.exp(m2 - m1)
    denom = 1.0 + e2
    w_ref[...] = jnp.concatenate([1.0 / denom, e2 / denom], axis=-1)
    i_ref[...] = jnp.concatenate([i1, i2], axis=-1)


def _gating(x, Wg, bg):
    grid = (N // BM,)
    return pl.pallas_call(
        _gating_body,
        grid=grid,
        in_specs=[
            pl.BlockSpec((BM, D), lambda i: (i, 0)),
            pl.BlockSpec((D, E), lambda i: (0, 0)),
            pl.BlockSpec((1, E), lambda i: (0, 0)),
        ],
        out_specs=[
            pl.BlockSpec((BM, K), lambda i: (i, 0)),
            pl.BlockSpec((BM, K), lambda i: (i, 0)),
        ],
        out_shape=[
            jax.ShapeDtypeStruct((N, K), jnp.float32),
            jax.ShapeDtypeStruct((N, K), jnp.int32),
        ],
    )(x, Wg, bg.reshape(1, E))


def _sc_gather_rows(x, token_ids):
    per_w = SLOTS // NW
    mesh = plsc.VectorSubcoreMesh(core_axis_name="c", subcore_axis_name="s")

    @functools.partial(
        pl.kernel, mesh=mesh,
        out_type=jax.ShapeDtypeStruct((SLOTS, D), jnp.float32),
        scratch_types=[
            pltpu.VMEM((per_w,), jnp.int32),
            pltpu.VMEM((per_w, D), jnp.float32),
            pltpu.SemaphoreType.DMA,
        ],
    )
    def k(x_hbm, idx_hbm, out_hbm, idx_v, rows_v, sem):
        wid = lax.axis_index("s") * 2 + lax.axis_index("c")
        base = wid * per_w
        pltpu.sync_copy(idx_hbm.at[pl.ds(base, per_w)], idx_v)
        pltpu.async_copy(x_hbm.at[idx_v], rows_v, sem).wait()
        pltpu.sync_copy(rows_v, out_hbm.at[pl.ds(base, per_w)])

    return k(x, token_ids)


def _mlp_body(tile_s, exp_s, lo_s, hi_s,
              xs_ref, w1_ref, b1_ref, w2_ref, b2_ref, gw_ref, out_ref):
    s = pl.program_id(0)
    lo, hi = lo_s[s], hi_s[s]

    @pl.when(hi > lo)
    def _():
        xb = xs_ref[...]
        h = jnp.dot(xb, w1_ref[0], preferred_element_type=jnp.float32)
        h = jnp.maximum(h + b1_ref[0], 0.0)
        y = jnp.dot(h, w2_ref[0], preferred_element_type=jnp.float32)
        y = (y + b2_ref[0]) * gw_ref[...]
        base = tile_s[s] * BM
        rows = base + lax.broadcasted_iota(jnp.int32, (BM, 1), 0)
        mask = (rows >= lo) & (rows < hi)
        out_ref[...] = jnp.where(mask, y, out_ref[...])


def _mlp_grouped(xs, W1, b1, W2, b2, gw, tile_s, exp_s, lo_s, hi_s):
    grid_spec = pltpu.PrefetchScalarGridSpec(
        num_scalar_prefetch=4,
        grid=(S,),
        in_specs=[
            pl.BlockSpec((BM, D), lambda s, t, e, lo, hi: (t[s], 0)),
            pl.BlockSpec((1, D, H), lambda s, t, e, lo, hi: (e[s], 0, 0)),
            pl.BlockSpec((1, 1, H), lambda s, t, e, lo, hi: (e[s], 0, 0)),
            pl.BlockSpec((1, H, O), lambda s, t, e, lo, hi: (e[s], 0, 0)),
            pl.BlockSpec((1, 1, O), lambda s, t, e, lo, hi: (e[s], 0, 0)),
            pl.BlockSpec((BM, 1), lambda s, t, e, lo, hi: (t[s], 0)),
        ],
        out_specs=pl.BlockSpec((BM, O), lambda s, t, e, lo, hi: (t[s], 0)),
    )
    return pl.pallas_call(
        _mlp_body,
        grid_spec=grid_spec,
        out_shape=jax.ShapeDtypeStruct((SLOTS, O), jnp.float32),
    )(tile_s, exp_s, lo_s, hi_s, xs, W1, b1.reshape(E, 1, H), W2,
      b2.reshape(E, 1, O), gw.reshape(SLOTS, 1))


def _sc_combine(ys, pos0, pos1):
    per_w = N // NW
    mesh = plsc.VectorSubcoreMesh(core_axis_name="c", subcore_axis_name="s")

    @functools.partial(
        pl.kernel, mesh=mesh,
        out_type=jax.ShapeDtypeStruct((N, O), jnp.float32),
        scratch_types=[
            pltpu.VMEM((per_w,), jnp.int32),
            pltpu.VMEM((per_w,), jnp.int32),
            pltpu.VMEM((per_w, O), jnp.float32),
            pltpu.VMEM((per_w, O), jnp.float32),
            pltpu.SemaphoreType.DMA,
            pltpu.SemaphoreType.DMA,
        ],
    )
    def k(ys_hbm, p0_hbm, p1_hbm, out_hbm, i0, i1, r0, r1, sem0, sem1):
        wid = lax.axis_index("s") * 2 + lax.axis_index("c")
        base = wid * per_w
        pltpu.sync_copy(p0_hbm.at[pl.ds(base, per_w)], i0)
        pltpu.sync_copy(p1_hbm.at[pl.ds(base, per_w)], i1)
        c0 = pltpu.async_copy(ys_hbm.at[i0], r0, sem0)
        c1 = pltpu.async_copy(ys_hbm.at[i1], r1, sem1)
        c0.wait()
        c1.wait()

        @pl.loop(0, per_w)
        def _(j):
            @pl.loop(0, O, step=16)
            def _(c):
                r0[j, pl.ds(c, 16)] = r0[j, pl.ds(c, 16)] + r1[j, pl.ds(c, 16)]

        pltpu.sync_copy(r0, out_hbm.at[pl.ds(base, per_w)])

    return k(ys, pos0, pos1)


def _route_metadata(e_flat):
    sort_idx = jnp.argsort(e_flat, stable=True).astype(jnp.int32)
    inv = jnp.zeros((SLOTS,), jnp.int32).at[sort_idx].set(
        jnp.arange(SLOTS, dtype=jnp.int32))
    counts = jnp.bincount(e_flat, length=E)
    offsets = jnp.concatenate(
        [jnp.zeros((1,), counts.dtype), jnp.cumsum(counts)]).astype(jnp.int32)

    t_all = jnp.repeat(jnp.arange(T, dtype=jnp.int32), E)
    e_all = jnp.tile(jnp.arange(E, dtype=jnp.int32), T)
    lo_all = jnp.maximum(t_all * BM, offsets[e_all])
    hi_all = jnp.minimum((t_all + 1) * BM, offsets[e_all + 1])
    valid = hi_all > lo_all
    key = jnp.where(valid, t_all * E + e_all, jnp.int32(T * E + 1))
    order = jnp.argsort(key)[:S]
    t_s, e_s = t_all[order], e_all[order]
    lo_s, hi_s = lo_all[order], hi_all[order]
    nv = jnp.sum(valid.astype(jnp.int32))
    last = jnp.maximum(nv - 1, 0)
    pad = jnp.arange(S, dtype=jnp.int32) >= nv
    t_s = jnp.where(pad, t_s[last], t_s)
    e_s = jnp.where(pad, e_s[last], e_s)
    lo_s = jnp.where(pad, 0, lo_s)
    hi_s = jnp.where(pad, 0, hi_s)
    return sort_idx, inv, t_s, e_s, lo_s, hi_s


def kernel(x, Wg, bg, W1, b1, W2, b2):
    gate_w, gate_i = _gating(x, Wg, bg)

    e_flat = gate_i.reshape(SLOTS)
    sort_idx, inv, t_s, e_s, lo_s, hi_s = _route_metadata(e_flat)
    token_ids = sort_idx // K
    gw_sorted = gate_w.reshape(SLOTS)[sort_idx]

    xs = _sc_gather_rows(x, token_ids)
    ys = _mlp_grouped(xs, W1, b1, W2, b2, gw_sorted, t_s, e_s, lo_s, hi_s)

    pos = inv.reshape(N, K)
    return _sc_combine(ys, pos[:, 0], pos[:, 1])
